# Initial kernel scaffold; baseline (speedup 1.0000x reference)
import jax
import jax.numpy as jnp
from jax.experimental import pallas as pl


def kernel(x, edge_index, edge_attr, pos, batch, W1, root1, b1, W2, root2, b2, fc1_w, fc1_b, fc2_w, fc2_b):
    raise NotImplementedError("write your pallas kernel here")


# speedup vs baseline: 1.1252x; 1.1252x over previous
import functools

import jax
import jax.numpy as jnp
import numpy as np
from jax.experimental import pallas as pl
from jax.experimental.pallas import tpu as pltpu

G = 16
M = 3


def _mm_body(x_ref, w_ref, o_ref):
    o_ref[...] = jnp.dot(x_ref[...], w_ref[...],
                         preferred_element_type=jnp.float32)


def _pallas_matmul(x, w, block_rows=256):
    N, K = x.shape
    K2, C = w.shape
    assert K == K2
    Np = (N + block_rows - 1) // block_rows * block_rows
    Kp = (K + 127) // 128 * 128
    Cp = (C + 127) // 128 * 128
    xp = jnp.pad(x, ((0, Np - N), (0, Kp - K)))
    wp = jnp.pad(w, ((0, Kp - K), (0, Cp - C)))
    out = pl.pallas_call(
        _mm_body,
        grid=(Np // block_rows,),
        in_specs=[
            pl.BlockSpec((block_rows, Kp), lambda i: (i, 0)),
            pl.BlockSpec((Kp, Cp), lambda i: (0, 0)),
        ],
        out_specs=pl.BlockSpec((block_rows, Cp), lambda i: (i, 0)),
        out_shape=jax.ShapeDtypeStruct((Np, Cp), jnp.float32),
    )(xp, wp)
    return out[:N, :C]


def _spline_basis(pseudo):
    E = pseudo.shape[0]
    v = jnp.clip(pseudo, 0.0, 1.0) * (M - 1)
    lo = jnp.clip(jnp.floor(v), 0, M - 2)
    fr = v - lo
    lo = lo.astype(jnp.int32)
    B = jnp.zeros((E, M ** 3), dtype=pseudo.dtype)
    e = jnp.arange(E)
    for b0 in (0, 1):
        for b1 in (0, 1):
            for b2 in (0, 1):
                idx = (lo[:, 0] + b0) + M * (lo[:, 1] + b1) + M * M * (lo[:, 2] + b2)
                w0 = fr[:, 0] if b0 else 1.0 - fr[:, 0]
                w1 = fr[:, 1] if b1 else 1.0 - fr[:, 1]
                w2 = fr[:, 2] if b2 else 1.0 - fr[:, 2]
                B = B.at[e, idx].add(w0 * w1 * w2)
    return B


def _spline_conv(x, edge_index, pseudo, W, root, bias):
    N = x.shape[0]
    K, Cin, Cout = W.shape
    row, col = edge_index[0], edge_index[1]
    mask = row != col
    B = _spline_basis(pseudo)
    Wf = jnp.transpose(W, (1, 0, 2)).reshape(Cin, K * Cout)
    Z = _pallas_matmul(x, jnp.concatenate([Wf, root], axis=1))
    zr, root_term = Z[:, :K * Cout], Z[:, K * Cout:]
    y = jnp.take(zr, row, axis=0).reshape(-1, K, Cout)
    msg = jnp.sum(B[:, :, None] * y, axis=1)
    msg = msg * mask[:, None].astype(msg.dtype)
    deg = jax.ops.segment_sum(mask.astype(x.dtype), col, num_segments=N)
    agg = jax.ops.segment_sum(msg, col, num_segments=N) / jnp.maximum(deg, 1.0)[:, None]
    return agg + root_term + bias


def _normalized_cut(edge_index, pos):
    N = pos.shape[0]
    row, col = edge_index[0], edge_index[1]
    diff = pos[row] - pos[col]
    d = jnp.sqrt(jnp.maximum(jnp.sum(diff * diff, axis=1), 1e-12))
    deg = jax.ops.segment_sum(jnp.ones_like(d), col, num_segments=N)
    inv = jnp.where(deg > 0, 1.0 / jnp.maximum(deg, 1.0), 0.0)
    return d * (jnp.take(inv, row) + jnp.take(inv, col))


def _graclus(edge_index, weight, N):
    row, col = edge_index[0], edge_index[1]
    mask = row != col
    wm = jnp.where(mask, weight, -jnp.inf)
    wmax = jax.ops.segment_max(wm, row, num_segments=N)
    isbest = mask & (wm >= jnp.take(wmax, row)) & jnp.isfinite(wm)
    cand = jnp.where(isbest, col, -1)
    best = jax.ops.segment_max(cand, row, num_segments=N)
    best = jnp.maximum(best, -1)
    safe = jnp.clip(best, 0, N - 1)
    idx = jnp.arange(N)
    mutual = (best >= 0) & (jnp.take(best, safe) == idx)
    return jnp.where(mutual, jnp.minimum(idx, safe), idx)


def _pool_step(cluster, x, pos, batch):
    N = x.shape[0]
    cnt = jax.ops.segment_sum(jnp.ones((N,), x.dtype), cluster, num_segments=N)
    xm = jax.ops.segment_max(x, cluster, num_segments=N)
    xp = jnp.where((cnt > 0)[:, None], xm, 0.0)
    posp = jax.ops.segment_sum(pos, cluster, num_segments=N) / jnp.maximum(cnt, 1.0)[:, None]
    bm = jax.ops.segment_max(batch, cluster, num_segments=N)
    bp = jnp.where(cnt > 0, bm, G)
    return xp, posp, bp


def kernel(x, edge_index, edge_attr, pos, batch, W1, root1, b1, W2, root2, b2,
           fc1_w, fc1_b, fc2_w, fc2_b):
    N = x.shape[0]
    h = jax.nn.elu(_spline_conv(x, edge_index, edge_attr, W1, root1, b1))
    w = _normalized_cut(edge_index, pos)
    c1 = _graclus(edge_index, w, N)
    h, pos2, batch2 = _pool_step(c1, h, pos, batch)
    ei2 = jnp.take(c1, edge_index)
    h = jax.nn.elu(_spline_conv(h, ei2, edge_attr, W2, root2, b2))
    w2 = _normalized_cut(ei2, pos2)
    c2 = _graclus(ei2, w2, N)
    cnt2 = jax.ops.segment_sum(jnp.ones((N,), h.dtype), c2, num_segments=N)
    xm = jax.ops.segment_max(h, c2, num_segments=N)
    xp = jnp.where((cnt2 > 0)[:, None], xm, 0.0)
    bm = jax.ops.segment_max(batch2, c2, num_segments=N)
    bp = jnp.where(cnt2 > 0, bm, G)
    sums = jax.ops.segment_sum(xp, bp, num_segments=G + 1)
    cnts = jax.ops.segment_sum(jnp.ones((N,), h.dtype), bp, num_segments=G + 1)
    gx = sums[:G] / jnp.maximum(cnts[:G], 1.0)[:, None]
    out = jax.nn.elu(gx @ fc1_w + fc1_b)
    out = jax.nn.elu(out @ fc2_w + fc2_b)
    return out
